# baseline (speedup 1.0000x reference)
import functools

import jax
import jax.numpy as jnp
from jax.experimental import pallas as pl
from jax.experimental.pallas import tpu as pltpu

_BT = 512


def _router_kernel(x_ref, m_ref, wt_ref, b_ref,
                   logits_ref, tki_ref, tkw_ref, aux_ref,
                   imp_acc, *, target_load):
    i = pl.program_id(0)
    n = pl.num_programs(0)

    logits = jax.lax.dot_general(
        x_ref[...], wt_ref[...], (((1,), (0,)), ((), ())),
        preferred_element_type=jnp.float32,
        precision=jax.lax.Precision.DEFAULT)
    logits = logits + m_ref[...] * b_ref[...]
    logits_ref[...] = logits

    mx = jnp.max(logits, axis=1, keepdims=True)
    e = jnp.exp(logits - mx)
    s = jnp.sum(e, axis=1, keepdims=True)
    probs = e / s

    iota = jax.lax.broadcasted_iota(jnp.int32, probs.shape, 1)
    big = jnp.int32(2**30)
    v1 = jnp.max(probs, axis=1, keepdims=True)
    i1 = jnp.min(jnp.where(probs == v1, iota, big), axis=1, keepdims=True)
    probs2 = jnp.where(iota == i1, jnp.float32(-1.0), probs)
    v2 = jnp.max(probs2, axis=1, keepdims=True)
    i2 = jnp.min(jnp.where(probs2 == v2, iota, big), axis=1, keepdims=True)
    tkw_ref[...] = jnp.concatenate([v1, v2], axis=1)
    tki_ref[...] = jnp.concatenate([i1, i2], axis=1)

    part = jnp.sum(probs, axis=0, keepdims=True)

    @pl.when(i == 0)
    def _():
        imp_acc[...] = part

    @pl.when(i > 0)
    def _():
        imp_acc[...] += part

    @pl.when(i == n - 1)
    def _():
        imp = imp_acc[...]
        aux_ref[...] = jnp.mean((imp - target_load) ** 2).reshape(1, 1)


def kernel(hidden_states, mass, W, mass_bias):
    B, T, C = hidden_states.shape
    E = W.shape[0]
    N = B * T
    x = hidden_states.reshape(N, C)
    m = mass.reshape(N, 1)
    wt = W.T
    b = mass_bias.reshape(1, E)
    grid = N // _BT

    kfn = functools.partial(_router_kernel, target_load=float(N) / float(E))
    logits, tki, tkw, aux = pl.pallas_call(
        kfn,
        grid=(grid,),
        in_specs=[
            pl.BlockSpec((_BT, C), lambda i: (i, 0)),
            pl.BlockSpec((_BT, 1), lambda i: (i, 0)),
            pl.BlockSpec((C, E), lambda i: (0, 0)),
            pl.BlockSpec((1, E), lambda i: (0, 0)),
        ],
        out_specs=[
            pl.BlockSpec((_BT, E), lambda i: (i, 0)),
            pl.BlockSpec((_BT, 2), lambda i: (i, 0)),
            pl.BlockSpec((_BT, 2), lambda i: (i, 0)),
            pl.BlockSpec((1, 1), lambda i: (0, 0)),
        ],
        out_shape=[
            jax.ShapeDtypeStruct((N, E), jnp.float32),
            jax.ShapeDtypeStruct((N, 2), jnp.int32),
            jax.ShapeDtypeStruct((N, 2), jnp.float32),
            jax.ShapeDtypeStruct((1, 1), jnp.float32),
        ],
        scratch_shapes=[pltpu.VMEM((1, E), jnp.float32)],
    )(x, m, wt, b)
    return (logits, tki, aux.reshape(()), tkw)


# speedup vs baseline: 1.1097x; 1.1097x over previous
import functools

import jax
import jax.numpy as jnp
from jax.experimental import pallas as pl
from jax.experimental.pallas import tpu as pltpu

_BT = 1024


def _router_kernel(x_ref, m_ref, wt_ref, b_ref,
                   logits_ref, tki_ref, tkw_ref, aux_ref,
                   imp_acc, *, target_load):
    i = pl.program_id(0)
    n = pl.num_programs(0)

    logits = jax.lax.dot_general(
        x_ref[...], wt_ref[...], (((1,), (0,)), ((), ())),
        preferred_element_type=jnp.float32,
        precision=jax.lax.Precision.DEFAULT)
    logits = logits + m_ref[...] * b_ref[...]
    logits_ref[...] = logits

    mx = jnp.max(logits, axis=1, keepdims=True)
    e = jnp.exp(logits - mx)
    s = jnp.sum(e, axis=1, keepdims=True)
    probs = e / s

    iota = jax.lax.broadcasted_iota(jnp.int32, probs.shape, 1)
    big = jnp.int32(2**30)
    v1 = jnp.max(probs, axis=1, keepdims=True)
    i1 = jnp.min(jnp.where(probs == v1, iota, big), axis=1, keepdims=True)
    probs2 = jnp.where(iota == i1, jnp.float32(-1.0), probs)
    v2 = jnp.max(probs2, axis=1, keepdims=True)
    i2 = jnp.min(jnp.where(probs2 == v2, iota, big), axis=1, keepdims=True)
    tkw_ref[...] = jnp.concatenate([v1, v2], axis=1)
    tki_ref[...] = jnp.concatenate([i1, i2], axis=1)

    part = jnp.sum(probs, axis=0, keepdims=True)

    @pl.when(i == 0)
    def _():
        imp_acc[...] = part

    @pl.when(i > 0)
    def _():
        imp_acc[...] += part

    @pl.when(i == n - 1)
    def _():
        imp = imp_acc[...]
        aux_ref[...] = jnp.mean((imp - target_load) ** 2).reshape(1, 1)


def kernel(hidden_states, mass, W, mass_bias):
    B, T, C = hidden_states.shape
    E = W.shape[0]
    N = B * T
    x = hidden_states.reshape(N, C)
    m = mass.reshape(N, 1)
    wt = W.T
    b = mass_bias.reshape(1, E)
    grid = N // _BT

    kfn = functools.partial(_router_kernel, target_load=float(N) / float(E))
    logits, tki, tkw, aux = pl.pallas_call(
        kfn,
        grid=(grid,),
        in_specs=[
            pl.BlockSpec((_BT, C), lambda i: (i, 0)),
            pl.BlockSpec((_BT, 1), lambda i: (i, 0)),
            pl.BlockSpec((C, E), lambda i: (0, 0)),
            pl.BlockSpec((1, E), lambda i: (0, 0)),
        ],
        out_specs=[
            pl.BlockSpec((_BT, E), lambda i: (i, 0)),
            pl.BlockSpec((_BT, 2), lambda i: (i, 0)),
            pl.BlockSpec((_BT, 2), lambda i: (i, 0)),
            pl.BlockSpec((1, 1), lambda i: (0, 0)),
        ],
        out_shape=[
            jax.ShapeDtypeStruct((N, E), jnp.float32),
            jax.ShapeDtypeStruct((N, 2), jnp.int32),
            jax.ShapeDtypeStruct((N, 2), jnp.float32),
            jax.ShapeDtypeStruct((1, 1), jnp.float32),
        ],
        scratch_shapes=[pltpu.VMEM((1, E), jnp.float32)],
    )(x, m, wt, b)
    return (logits, tki, aux.reshape(()), tkw)
